# Initial kernel scaffold; baseline (speedup 1.0000x reference)
import jax
import jax.numpy as jnp
from jax.experimental import pallas as pl


def kernel(target, logits, step, summary_step):
    raise NotImplementedError("write your pallas kernel here")


# speedup vs baseline: 1.0102x; 1.0102x over previous
import functools

import jax
import jax.numpy as jnp
from jax.experimental import pallas as pl
from jax.experimental.pallas import tpu as pltpu

_KS = (1832780943, 270669613)
_KU = (64467757, 2916123636)
_NEG_INF = -1.0e9
_TINY = 1.1754943508222875e-38
_INT_MAX = 2147483647


def _threefry2x32(k0, k1, x0, x1):
    ks0 = jnp.uint32(k0)
    ks1 = jnp.uint32(k1)
    ks2 = jnp.uint32(k0 ^ k1 ^ 0x1BD11BDA)
    ks = (ks0, ks1, ks2)
    rot = ((13, 15, 26, 6), (17, 29, 16, 24))
    x0 = x0 + ks0
    x1 = x1 + ks1
    for g in range(5):
        for r in rot[g % 2]:
            x0 = x0 + x1
            x1 = (x1 << r) | (x1 >> (32 - r))
            x1 = x1 ^ x0
        x0 = x0 + ks[(g + 1) % 3]
        x1 = x1 + ks[(g + 2) % 3] + jnp.uint32(g + 1)
    return x0, x1


def _random_u01(key, flat_idx_u32):
    b0, b1 = _threefry2x32(key[0], key[1], jnp.zeros_like(flat_idx_u32), flat_idx_u32)
    bits = b0 ^ b1
    fbits = (bits >> 9) | jnp.uint32(0x3F800000)
    return jax.lax.bitcast_convert_type(fbits, jnp.float32) - jnp.float32(1.0)


def _body(step_ref, thr_ref, tgt_ref, x_ref, out_ref,
          m_ref, s_ref, bv_ref, bi_ref, xt_ref, tc_ref, *, B, L, V, BV, NV):
    j = pl.program_id(0)

    @pl.when(j == 0)
    def _init():
        m_ref[...] = jnp.full((B, 1), -jnp.inf, jnp.float32)
        s_ref[...] = jnp.zeros((B, 1), jnp.float32)
        bv_ref[...] = jnp.full((B, 1), -jnp.inf, jnp.float32)
        bi_ref[...] = jnp.zeros((B, 1), jnp.int32)
        xt_ref[...] = jnp.zeros((B, 1), jnp.float32)
        colL = jax.lax.broadcasted_iota(jnp.int32, (B, L), 1)
        tc_ref[...] = jnp.sum(jnp.where(colL == step_ref[0], tgt_ref[...], 0),
                              axis=1, keepdims=True)

    x = x_ref[...]
    row = jax.lax.broadcasted_iota(jnp.int32, (B, BV), 0)
    col = jax.lax.broadcasted_iota(jnp.int32, (B, BV), 1) + j * BV
    valid = col < V

    cnt = (row * V + col).astype(jnp.uint32)
    f = _random_u01(_KS, cnt)
    u = jnp.maximum(jnp.float32(_TINY),
                    f * jnp.float32(1.0 - _TINY) + jnp.float32(_TINY))
    g = -jnp.log(-jnp.log(u))

    val = jnp.where(valid, x + g, -jnp.inf)
    lm = jnp.max(val, axis=1, keepdims=True)
    li = jnp.min(jnp.where(val == lm, col, _INT_MAX), axis=1, keepdims=True)
    upd = lm > bv_ref[...]
    bi_ref[...] = jnp.where(upd, li, bi_ref[...])
    bv_ref[...] = jnp.where(upd, lm, bv_ref[...])

    xm = jnp.where(valid, x, -jnp.inf)
    bm = jnp.max(xm, axis=1, keepdims=True)
    m_old = m_ref[...]
    m_new = jnp.maximum(m_old, bm)
    e = jnp.where(valid, jnp.exp(x - m_new), jnp.float32(0.0))
    s_ref[...] = s_ref[...] * jnp.exp(m_old - m_new) + jnp.sum(e, axis=1, keepdims=True)
    m_ref[...] = m_new

    xt_ref[...] += jnp.sum(jnp.where(col == tc_ref[...], x, jnp.float32(0.0)),
                           axis=1, keepdims=True)

    @pl.when(j == NV - 1)
    def _fin():
        t = tgt_ref[...]
        colL = jax.lax.broadcasted_iota(jnp.int32, (B, L), 1)
        rowL = jax.lax.broadcasted_iota(jnp.int32, (B, L), 0)
        uL = _random_u01(_KU, (rowL * L + colL).astype(jnp.uint32))
        maskf = (t > 0).astype(jnp.float32)
        rv = uL + (jnp.float32(1.0) - maskf) * jnp.float32(_NEG_INF)
        lmL = jnp.max(rv, axis=1, keepdims=True)
        riL = jnp.min(jnp.where(rv == lmL, colL, _INT_MAX), axis=1, keepdims=True)
        rand_tgt = jnp.sum(jnp.where(colL == riL, t, 0), axis=1, keepdims=True)
        gt = tc_ref[...]
        conf = jnp.exp(xt_ref[...] - m_ref[...]) / s_ref[...]
        gold = thr_ref[0]
        randp = thr_ref[1]
        sel = jnp.where(conf < gold, gt,
                        jnp.where(conf < randp, bi_ref[...], rand_tgt))
        out_ref[...] = sel.astype(jnp.int32)


def kernel(target, logits, step, summary_step):
    del summary_step
    B, L = target.shape
    _, V = logits.shape
    BV = 2048
    NV = pl.cdiv(V, BV)

    stepi = jnp.asarray(step, jnp.int32).reshape((1,))
    gold = jnp.float32(0.9) * jnp.exp(-jnp.asarray(step, jnp.float32) / 20000.0)
    randp = gold + jnp.float32(0.5) * (jnp.float32(1.0) - gold)
    thr = jnp.stack([gold, randp]).astype(jnp.float32)

    body = functools.partial(_body, B=B, L=L, V=V, BV=BV, NV=NV)
    out = pl.pallas_call(
        body,
        grid=(NV,),
        in_specs=[
            pl.BlockSpec(memory_space=pltpu.SMEM),
            pl.BlockSpec(memory_space=pltpu.SMEM),
            pl.BlockSpec((B, L), lambda j: (0, 0)),
            pl.BlockSpec((B, BV), lambda j: (0, j)),
        ],
        out_specs=pl.BlockSpec((B, 1), lambda j: (0, 0)),
        out_shape=jax.ShapeDtypeStruct((B, 1), jnp.int32),
        scratch_shapes=[
            pltpu.VMEM((B, 1), jnp.float32),
            pltpu.VMEM((B, 1), jnp.float32),
            pltpu.VMEM((B, 1), jnp.float32),
            pltpu.VMEM((B, 1), jnp.int32),
            pltpu.VMEM((B, 1), jnp.float32),
            pltpu.VMEM((B, 1), jnp.int32),
        ],
        compiler_params=pltpu.CompilerParams(
            dimension_semantics=("arbitrary",),
        ),
    )(stepi, thr, target, logits)
    return out.astype(target.dtype)


# speedup vs baseline: 1.0156x; 1.0054x over previous
import functools

import jax
import jax.numpy as jnp
from jax.experimental import pallas as pl
from jax.experimental.pallas import tpu as pltpu

_KS = (1832780943, 270669613)
_KU = (64467757, 2916123636)
_NEG_INF = -1.0e9
_TINY = 1.1754943508222875e-38
_INT_MAX = 2147483647


def _threefry2x32(k0, k1, x0, x1):
    ks0 = jnp.uint32(k0)
    ks1 = jnp.uint32(k1)
    ks2 = jnp.uint32(k0 ^ k1 ^ 0x1BD11BDA)
    ks = (ks0, ks1, ks2)
    rot = ((13, 15, 26, 6), (17, 29, 16, 24))
    x0 = x0 + ks0
    x1 = x1 + ks1
    for g in range(5):
        for r in rot[g % 2]:
            x0 = x0 + x1
            x1 = (x1 << r) | (x1 >> (32 - r))
            x1 = x1 ^ x0
        x0 = x0 + ks[(g + 1) % 3]
        x1 = x1 + ks[(g + 2) % 3] + jnp.uint32(g + 1)
    return x0, x1


def _random_u01(key, flat_idx_u32):
    b0, b1 = _threefry2x32(key[0], key[1], jnp.zeros_like(flat_idx_u32), flat_idx_u32)
    bits = b0 ^ b1
    fbits = (bits >> 9) | jnp.uint32(0x3F800000)
    return jax.lax.bitcast_convert_type(fbits, jnp.float32) - jnp.float32(1.0)


def _gumbel(flat_idx_u32):
    f = _random_u01(_KS, flat_idx_u32)
    u = jnp.maximum(jnp.float32(_TINY),
                    f * jnp.float32(1.0 - _TINY) + jnp.float32(_TINY))
    return -jnp.log(-jnp.log(u))


def _body(step_ref, thr_ref, tgt_ref, x_ref, out_ref,
          m_ref, s_ref, bv_ref, bi_ref, xt_ref, tc_ref, cb_ref,
          *, B, L, V, BV, NV):
    j = pl.program_id(0)

    @pl.when(j == 0)
    def _init():
        m_ref[...] = jnp.full((B, 1), -jnp.inf, jnp.float32)
        s_ref[...] = jnp.zeros((B, 1), jnp.float32)
        bv_ref[...] = jnp.full((B, 1), -jnp.inf, jnp.float32)
        bi_ref[...] = jnp.zeros((B, 1), jnp.int32)
        xt_ref[...] = jnp.zeros((B, 1), jnp.float32)
        colL = jax.lax.broadcasted_iota(jnp.int32, (B, L), 1)
        tc_ref[...] = jnp.sum(jnp.where(colL == step_ref[0], tgt_ref[...], 0),
                              axis=1, keepdims=True)
        row = jax.lax.broadcasted_iota(jnp.int32, (B, BV), 0)
        colb = jax.lax.broadcasted_iota(jnp.int32, (B, BV), 1)
        cb_ref[...] = row * V + colb

    def sweep(masked):
        x = x_ref[...]
        col0 = j * BV
        cnt = (cb_ref[...] + col0).astype(jnp.uint32)
        g = _gumbel(cnt)
        val = x + g
        if masked:
            col = jax.lax.broadcasted_iota(jnp.int32, (B, BV), 1) + col0
            valid = col < V
            val = jnp.where(valid, val, -jnp.inf)
        lm = jnp.max(val, axis=1, keepdims=True)
        eqcol = jax.lax.broadcasted_iota(jnp.int32, (B, BV), 1)
        li = col0 + jnp.min(jnp.where(val == lm, eqcol, _INT_MAX),
                            axis=1, keepdims=True)
        upd = lm > bv_ref[...]
        bi_ref[...] = jnp.where(upd, li, bi_ref[...])
        bv_ref[...] = jnp.where(upd, lm, bv_ref[...])

        xm = jnp.where(valid, x, -jnp.inf) if masked else x
        bm = jnp.max(xm, axis=1, keepdims=True)
        m_old = m_ref[...]
        m_new = jnp.maximum(m_old, bm)
        e = jnp.exp(x - m_new)
        if masked:
            e = jnp.where(valid, e, jnp.float32(0.0))
        s_ref[...] = (s_ref[...] * jnp.exp(m_old - m_new)
                      + jnp.sum(e, axis=1, keepdims=True))
        m_ref[...] = m_new

        xt_ref[...] += jnp.sum(
            jnp.where(eqcol == tc_ref[...] - col0, x, jnp.float32(0.0)),
            axis=1, keepdims=True)

    @pl.when(j < NV - 1)
    def _main():
        sweep(masked=False)

    @pl.when(j == NV - 1)
    def _last():
        sweep(masked=True)

        t = tgt_ref[...]
        colL = jax.lax.broadcasted_iota(jnp.int32, (B, L), 1)
        rowL = jax.lax.broadcasted_iota(jnp.int32, (B, L), 0)
        uL = _random_u01(_KU, (rowL * L + colL).astype(jnp.uint32))
        maskf = (t > 0).astype(jnp.float32)
        rv = uL + (jnp.float32(1.0) - maskf) * jnp.float32(_NEG_INF)
        lmL = jnp.max(rv, axis=1, keepdims=True)
        riL = jnp.min(jnp.where(rv == lmL, colL, _INT_MAX), axis=1, keepdims=True)
        rand_tgt = jnp.sum(jnp.where(colL == riL, t, 0), axis=1, keepdims=True)
        conf = jnp.exp(xt_ref[...] - m_ref[...]) / s_ref[...]
        sel = jnp.where(conf < thr_ref[0], tc_ref[...],
                        jnp.where(conf < thr_ref[1], bi_ref[...], rand_tgt))
        out_ref[...] = sel.astype(jnp.int32)


def kernel(target, logits, step, summary_step):
    del summary_step
    B, L = target.shape
    _, V = logits.shape
    BV = 2048
    NV = pl.cdiv(V, BV)

    stepi = jnp.asarray(step, jnp.int32).reshape((1,))
    gold = jnp.float32(0.9) * jnp.exp(-jnp.asarray(step, jnp.float32) / 20000.0)
    randp = gold + jnp.float32(0.5) * (jnp.float32(1.0) - gold)
    thr = jnp.stack([gold, randp]).astype(jnp.float32)

    body = functools.partial(_body, B=B, L=L, V=V, BV=BV, NV=NV)
    out = pl.pallas_call(
        body,
        grid=(NV,),
        in_specs=[
            pl.BlockSpec(memory_space=pltpu.SMEM),
            pl.BlockSpec(memory_space=pltpu.SMEM),
            pl.BlockSpec((B, L), lambda j: (0, 0)),
            pl.BlockSpec((B, BV), lambda j: (0, j)),
        ],
        out_specs=pl.BlockSpec((B, 1), lambda j: (0, 0)),
        out_shape=jax.ShapeDtypeStruct((B, 1), jnp.int32),
        scratch_shapes=[
            pltpu.VMEM((B, 1), jnp.float32),
            pltpu.VMEM((B, 1), jnp.float32),
            pltpu.VMEM((B, 1), jnp.float32),
            pltpu.VMEM((B, 1), jnp.int32),
            pltpu.VMEM((B, 1), jnp.float32),
            pltpu.VMEM((B, 1), jnp.int32),
            pltpu.VMEM((B, BV), jnp.int32),
        ],
        compiler_params=pltpu.CompilerParams(
            dimension_semantics=("arbitrary",),
        ),
    )(stepi, thr, target, logits)
    return out.astype(target.dtype)
